# Initial kernel scaffold; baseline (speedup 1.0000x reference)
import jax
import jax.numpy as jnp
from jax.experimental import pallas as pl


def kernel(x, edge_index, batch, W1, b1, W2, b2):
    raise NotImplementedError("write your pallas kernel here")


# speedup vs baseline: 13.0717x; 13.0717x over previous
import jax
import jax.numpy as jnp
from jax import lax
from jax.experimental import pallas as pl
from jax.experimental.pallas import tpu as pltpu
from jax.experimental.pallas import tpu_sc as plsc

N = 10000
E = 320000
D = 128
OUT = 5
G = 20

NC = 2
NS = 16
NW = NC * NS
CHUNK = 128
CW = 80
EP = NW * CW * CHUNK
NP = 10240
DUMMY = N
RPT = NP // NS
GP = 32
BLK = 1024


def _mesh():
    return plsc.VectorSubcoreMesh(
        core_axis_name="c", subcore_axis_name="s",
        num_cores=NC, num_subcores=NS)


def _sc_degree(dstp4, ones_chunk, zeros_w):
    def body(dst_hbm, ones_hbm, zeros_hbm, out_hbm, didx_v, ones_v, acc_sh):
        c = lax.axis_index("c")
        s = lax.axis_index("s")
        wid = s * NC + c
        t0 = s * RPT
        pltpu.sync_copy(zeros_hbm, acc_sh.at[pl.ds(t0, RPT)])
        pltpu.sync_copy(ones_hbm, ones_v)
        pltpu.sync_copy(dst_hbm.at[wid], didx_v)
        plsc.subcore_barrier()

        def step(k, carry):
            pltpu.sync_copy(ones_v, acc_sh.at[didx_v.at[k]], add=True)
            return carry

        lax.fori_loop(0, CW, step, 0)
        plsc.subcore_barrier()
        pltpu.sync_copy(acc_sh.at[pl.ds(t0, RPT)],
                        out_hbm.at[c, pl.ds(t0, RPT)])

    return pl.kernel(
        body,
        out_type=jax.ShapeDtypeStruct((NC, NP, 16), jnp.float32),
        mesh=_mesh(),
        compiler_params=pltpu.CompilerParams(use_tc_tiling_on_sc=False),
        scratch_types=[
            pltpu.VMEM((CW, CHUNK), jnp.int32),
            pltpu.VMEM((CHUNK, 16), jnp.float32),
            pltpu.VMEM_SHARED((NP, 16), jnp.float32),
        ],
    )(dstp4, ones_chunk, zeros_w)


def _sc_scatter_rows(table, srcp3, dstp4, zeros_w, width):
    def body(tab_hbm, src_hbm, dst_hbm, zeros_hbm, out_hbm,
             sidx_v, didx_v, rows_v, acc_sh, sem):
        c = lax.axis_index("c")
        s = lax.axis_index("s")
        wid = s * NC + c
        t0 = s * RPT
        pltpu.sync_copy(zeros_hbm, acc_sh.at[pl.ds(t0, RPT)])
        pltpu.sync_copy(src_hbm.at[wid], sidx_v)
        pltpu.sync_copy(dst_hbm.at[wid], didx_v)
        plsc.subcore_barrier()

        def step(k, carry):
            pltpu.async_copy(tab_hbm.at[sidx_v.at[k]], rows_v, sem).wait()
            pltpu.sync_copy(rows_v, acc_sh.at[didx_v.at[k]], add=True)
            return carry

        lax.fori_loop(0, CW, step, 0)
        plsc.subcore_barrier()
        pltpu.sync_copy(acc_sh.at[pl.ds(t0, RPT)],
                        out_hbm.at[c, pl.ds(t0, RPT)])

    return pl.kernel(
        body,
        out_type=jax.ShapeDtypeStruct((NC, NP, width), jnp.float32),
        mesh=_mesh(),
        compiler_params=pltpu.CompilerParams(
            use_tc_tiling_on_sc=False) if width < 128 else None,
        scratch_types=[
            pltpu.VMEM((CW, CHUNK), jnp.int32),
            pltpu.VMEM((CW, CHUNK), jnp.int32),
            pltpu.VMEM((CHUNK, width), jnp.float32),
            pltpu.VMEM_SHARED((NP, width), jnp.float32),
            pltpu.SemaphoreType.DMA,
        ],
    )(table, srcp3, dstp4, zeros_w)


def _dinv_from(deg_blk):
    return lax.rsqrt(deg_blk[0][:, 0:1] + deg_blk[1][:, 0:1] + 1.0)


def _tc_h1(xp, W1, deg2):
    def body(x_ref, w_ref, deg_ref, out_ref):
        dinv = _dinv_from(deg_ref)
        h = jnp.dot(x_ref[...], w_ref[...],
                    preferred_element_type=jnp.float32)
        out_ref[...] = h * dinv

    return pl.pallas_call(
        body,
        grid=(NP // BLK,),
        in_specs=[
            pl.BlockSpec((BLK, D), lambda i: (i, 0)),
            pl.BlockSpec((D, D), lambda i: (0, 0)),
            pl.BlockSpec((NC, BLK, 16), lambda i: (0, i, 0)),
        ],
        out_specs=pl.BlockSpec((BLK, D), lambda i: (i, 0)),
        out_shape=jax.ShapeDtypeStruct((NP, D), jnp.float32),
    )(xp, W1, deg2)


def _tc_z(agg, h1p, deg2, b1r, W2p):
    def body(agg_ref, h1_ref, deg_ref, b1_ref, w2_ref, out_ref):
        dinv = _dinv_from(deg_ref)
        pre = dinv * (agg_ref[0] + agg_ref[1] + h1_ref[...]) + b1_ref[...]
        h = jnp.where(pre >= 0, pre, 0.01 * pre)
        z = jnp.dot(h, w2_ref[...], preferred_element_type=jnp.float32)
        out_ref[...] = z * dinv

    return pl.pallas_call(
        body,
        grid=(NP // BLK,),
        in_specs=[
            pl.BlockSpec((NC, BLK, D), lambda i: (0, i, 0)),
            pl.BlockSpec((BLK, D), lambda i: (i, 0)),
            pl.BlockSpec((NC, BLK, 16), lambda i: (0, i, 0)),
            pl.BlockSpec((1, D), lambda i: (0, 0)),
            pl.BlockSpec((D, 16), lambda i: (0, 0)),
        ],
        out_specs=pl.BlockSpec((BLK, 16), lambda i: (i, 0)),
        out_shape=jax.ShapeDtypeStruct((NP, 16), jnp.float32),
    )(agg, h1p, deg2, b1r, W2p)


def _tc_pool(agg2, zp, deg2, batch2d, b2p):
    def body(agg_ref, zp_ref, deg_ref, b_ref, b2_ref, out_ref):
        dinv = _dinv_from(deg_ref)
        a2 = dinv * (agg_ref[0] + agg_ref[1] + zp_ref[...])
        gids = lax.broadcasted_iota(jnp.int32, (GP, NP), 0)
        mask = (b_ref[...] == gids).astype(jnp.float32)
        gsum = jnp.dot(mask, a2, preferred_element_type=jnp.float32)
        cnt = jnp.sum(mask, axis=1, keepdims=True)
        out_ref[...] = gsum / jnp.maximum(cnt, 1.0) + b2_ref[...]

    return pl.pallas_call(
        body,
        out_shape=jax.ShapeDtypeStruct((GP, 16), jnp.float32),
    )(agg2, zp, deg2, batch2d, b2p)


def kernel(x, edge_index, batch, W1, b1, W2, b2):
    x = x.astype(jnp.float32)
    src = edge_index[0].astype(jnp.int32)
    dst = edge_index[1].astype(jnp.int32)
    pad = EP - E
    srcp3 = jnp.concatenate(
        [src, jnp.zeros((pad,), jnp.int32)]).reshape(NW, CW, CHUNK)
    dstp4 = jnp.concatenate(
        [dst, jnp.full((pad,), DUMMY, jnp.int32)]).reshape(NW, CW, CHUNK)
    xp = jnp.concatenate([x, jnp.zeros((NP - N, D), jnp.float32)])
    batch2d = jnp.concatenate(
        [batch.astype(jnp.int32), jnp.full((NP - N,), G, jnp.int32)]
    ).reshape(1, NP)
    ones_chunk = jnp.ones((CHUNK, 16), jnp.float32)
    zeros16 = jnp.zeros((RPT, 16), jnp.float32)
    zeros128 = jnp.zeros((RPT, D), jnp.float32)
    W2p = jnp.concatenate(
        [W2.astype(jnp.float32), jnp.zeros((D, 16 - OUT), jnp.float32)], axis=1)
    b1r = b1.astype(jnp.float32).reshape(1, D)
    b2p = jnp.concatenate(
        [b2.astype(jnp.float32), jnp.zeros((16 - OUT,), jnp.float32)]
    ).reshape(1, 16)

    deg2 = _sc_degree(dstp4, ones_chunk, zeros16)
    h1p = _tc_h1(xp, W1.astype(jnp.float32), deg2)
    agg = _sc_scatter_rows(h1p, srcp3, dstp4, zeros128, D)
    zp = _tc_z(agg, h1p, deg2, b1r, W2p)
    agg2 = _sc_scatter_rows(zp, srcp3, dstp4, zeros16, 16)
    g = _tc_pool(agg2, zp, deg2, batch2d, b2p)
    g5 = g[:G, :OUT]
    return (g5[:, :2], g5[:, 2:4], g5[:, 4:5])


# speedup vs baseline: 16.4617x; 1.2593x over previous
import jax
import jax.numpy as jnp
from jax import lax
from jax.experimental import pallas as pl
from jax.experimental.pallas import tpu as pltpu
from jax.experimental.pallas import tpu_sc as plsc

N = 10000
E = 320000
D = 128
OUT = 5
G = 20

NC = 2
NS = 16
NW = NC * NS
CHUNK = 128
CW = 80
EP = NW * CW * CHUNK
NP = 10240
DUMMY = N
RPT = NP // NS
GP = 32
BLK = 1024
NB = 4
HW = 64
CWS = EP // (NS * CHUNK)


def _mesh():
    return plsc.VectorSubcoreMesh(
        core_axis_name="c", subcore_axis_name="s",
        num_cores=NC, num_subcores=NS)


def _sc_degree(dstp4, ones_chunk, zeros_w):
    def body(dst_hbm, ones_hbm, zeros_hbm, out_hbm, didx_v, ones_v, acc_sh):
        c = lax.axis_index("c")
        s = lax.axis_index("s")
        wid = s * NC + c
        t0 = s * RPT
        pltpu.sync_copy(zeros_hbm, acc_sh.at[pl.ds(t0, RPT)])
        pltpu.sync_copy(ones_hbm, ones_v)
        pltpu.sync_copy(dst_hbm.at[wid], didx_v)
        plsc.subcore_barrier()

        def step(k, carry):
            pltpu.sync_copy(ones_v, acc_sh.at[didx_v.at[k]], add=True)
            return carry

        lax.fori_loop(0, CW, step, 0)
        plsc.subcore_barrier()
        pltpu.sync_copy(acc_sh.at[pl.ds(t0, RPT)],
                        out_hbm.at[c, pl.ds(t0, RPT)])

    return pl.kernel(
        body,
        out_type=jax.ShapeDtypeStruct((NC, NP, 16), jnp.float32),
        mesh=_mesh(),
        compiler_params=pltpu.CompilerParams(use_tc_tiling_on_sc=False),
        scratch_types=[
            pltpu.VMEM((CW, CHUNK), jnp.int32),
            pltpu.VMEM((CHUNK, 16), jnp.float32),
            pltpu.VMEM_SHARED((NP, 16), jnp.float32),
        ],
    )(dstp4, ones_chunk, zeros_w)


def _sc_scatter_feat(tabs, srcp_s, dstp_s, zeros_w):
    def body(tab_hbm, src_hbm, dst_hbm, zeros_hbm, out_hbm,
             sidx_v, didx_v, r0, r1, r2, r3, acc_sh,
             g0, g1, g2, g3):
        rows_bufs = (r0, r1, r2, r3)
        gsems = (g0, g1, g2, g3)
        c = lax.axis_index("c")
        s = lax.axis_index("s")
        t0 = s * RPT
        pltpu.sync_copy(zeros_hbm, acc_sh.at[pl.ds(t0, RPT)])
        pltpu.sync_copy(src_hbm.at[s], sidx_v)
        pltpu.sync_copy(dst_hbm.at[s], didx_v)
        plsc.subcore_barrier()

        def group(g, carry):
            base = g * NB
            gds = []
            for b in range(NB):
                gds.append(pltpu.async_copy(
                    tab_hbm.at[c].at[sidx_v.at[base + b]],
                    rows_bufs[b], gsems[b]))
            for b in range(NB):
                gds[b].wait()
                pltpu.sync_copy(
                    rows_bufs[b], acc_sh.at[didx_v.at[base + b]], add=True)
            return carry

        lax.fori_loop(0, CWS // NB, group, 0)
        plsc.subcore_barrier()
        pltpu.sync_copy(acc_sh.at[pl.ds(t0, RPT)],
                        out_hbm.at[c, pl.ds(t0, RPT)])

    return pl.kernel(
        body,
        out_type=jax.ShapeDtypeStruct((NC, NP, HW), jnp.float32),
        mesh=_mesh(),
        compiler_params=pltpu.CompilerParams(use_tc_tiling_on_sc=False),
        scratch_types=[
            pltpu.VMEM((CWS, CHUNK), jnp.int32),
            pltpu.VMEM((CWS, CHUNK), jnp.int32),
            pltpu.VMEM((CHUNK, HW), jnp.float32),
            pltpu.VMEM((CHUNK, HW), jnp.float32),
            pltpu.VMEM((CHUNK, HW), jnp.float32),
            pltpu.VMEM((CHUNK, HW), jnp.float32),
            pltpu.VMEM_SHARED((NP, HW), jnp.float32),
            pltpu.SemaphoreType.DMA,
            pltpu.SemaphoreType.DMA,
            pltpu.SemaphoreType.DMA,
            pltpu.SemaphoreType.DMA,
        ],
    )(tabs, srcp_s, dstp_s, zeros_w)


def _sc_scatter_rows(table, srcp3, dstp4, zeros_w, width):
    def body(tab_hbm, src_hbm, dst_hbm, zeros_hbm, out_hbm,
             sidx_v, didx_v, r0, r1, r2, r3, acc_sh,
             g0, g1, g2, g3, s0, s1, s2, s3):
        rows_bufs = (r0, r1, r2, r3)
        gsems = (g0, g1, g2, g3)
        ssems = (s0, s1, s2, s3)
        c = lax.axis_index("c")
        s = lax.axis_index("s")
        wid = s * NC + c
        t0 = s * RPT
        pltpu.sync_copy(zeros_hbm, acc_sh.at[pl.ds(t0, RPT)])
        pltpu.sync_copy(src_hbm.at[wid], sidx_v)
        pltpu.sync_copy(dst_hbm.at[wid], didx_v)
        plsc.subcore_barrier()

        def group(g, carry):
            base = g * NB
            gds = []
            for b in range(NB):
                gds.append(pltpu.async_copy(
                    tab_hbm.at[sidx_v.at[base + b]], rows_bufs[b], gsems[b]))
            for b in range(NB):
                gds[b].wait()
                pltpu.sync_copy(
                    rows_bufs[b], acc_sh.at[didx_v.at[base + b]], add=True)
            return carry

        lax.fori_loop(0, CW // NB, group, 0)
        plsc.subcore_barrier()
        pltpu.sync_copy(acc_sh.at[pl.ds(t0, RPT)],
                        out_hbm.at[c, pl.ds(t0, RPT)])

    return pl.kernel(
        body,
        out_type=jax.ShapeDtypeStruct((NC, NP, width), jnp.float32),
        mesh=_mesh(),
        compiler_params=pltpu.CompilerParams(
            use_tc_tiling_on_sc=False) if width < 128 else None,
        scratch_types=[
            pltpu.VMEM((CW, CHUNK), jnp.int32),
            pltpu.VMEM((CW, CHUNK), jnp.int32),
            pltpu.VMEM((CHUNK, width), jnp.float32),
            pltpu.VMEM((CHUNK, width), jnp.float32),
            pltpu.VMEM((CHUNK, width), jnp.float32),
            pltpu.VMEM((CHUNK, width), jnp.float32),
            pltpu.VMEM_SHARED((NP, width), jnp.float32),
            pltpu.SemaphoreType.DMA,
            pltpu.SemaphoreType.DMA,
            pltpu.SemaphoreType.DMA,
            pltpu.SemaphoreType.DMA,
            pltpu.SemaphoreType.DMA,
            pltpu.SemaphoreType.DMA,
            pltpu.SemaphoreType.DMA,
            pltpu.SemaphoreType.DMA,
        ],
    )(table, srcp3, dstp4, zeros_w)


def _dinv_from(deg_blk):
    return lax.rsqrt(deg_blk[0][:, 0:1] + deg_blk[1][:, 0:1] + 1.0)


def _tc_h1(xp, W1, deg2):
    def body(x_ref, w_ref, deg_ref, out_ref):
        dinv = _dinv_from(deg_ref)
        h = jnp.dot(x_ref[...], w_ref[...],
                    preferred_element_type=jnp.float32)
        out_ref[...] = h * dinv

    return pl.pallas_call(
        body,
        grid=(NP // BLK,),
        in_specs=[
            pl.BlockSpec((BLK, D), lambda i: (i, 0)),
            pl.BlockSpec((D, D), lambda i: (0, 0)),
            pl.BlockSpec((NC, BLK, 16), lambda i: (0, i, 0)),
        ],
        out_specs=pl.BlockSpec((BLK, D), lambda i: (i, 0)),
        out_shape=jax.ShapeDtypeStruct((NP, D), jnp.float32),
    )(xp, W1, deg2)


def _tc_z(agg, h1p, deg2, b1r, W2p):
    def body(agg_ref, h1_ref, deg_ref, b1_ref, w2_ref, out_ref):
        dinv = _dinv_from(deg_ref)
        agg_full = jnp.concatenate([agg_ref[0], agg_ref[1]], axis=1)
        pre = dinv * (agg_full + h1_ref[...]) + b1_ref[...]
        h = jnp.where(pre >= 0, pre, 0.01 * pre)
        z = jnp.dot(h, w2_ref[...], preferred_element_type=jnp.float32)
        out_ref[...] = z * dinv

    return pl.pallas_call(
        body,
        grid=(NP // BLK,),
        in_specs=[
            pl.BlockSpec((NC, BLK, HW), lambda i: (0, i, 0)),
            pl.BlockSpec((BLK, D), lambda i: (i, 0)),
            pl.BlockSpec((NC, BLK, 16), lambda i: (0, i, 0)),
            pl.BlockSpec((1, D), lambda i: (0, 0)),
            pl.BlockSpec((D, 16), lambda i: (0, 0)),
        ],
        out_specs=pl.BlockSpec((BLK, 16), lambda i: (i, 0)),
        out_shape=jax.ShapeDtypeStruct((NP, 16), jnp.float32),
    )(agg, h1p, deg2, b1r, W2p)


def _tc_pool(agg2, zp, deg2, batch2d, b2p):
    def body(agg_ref, zp_ref, deg_ref, b_ref, b2_ref, out_ref):
        dinv = _dinv_from(deg_ref)
        a2 = dinv * (agg_ref[0] + agg_ref[1] + zp_ref[...])
        gids = lax.broadcasted_iota(jnp.int32, (GP, NP), 0)
        mask = (b_ref[...] == gids).astype(jnp.float32)
        gsum = jnp.dot(mask, a2, preferred_element_type=jnp.float32)
        cnt = jnp.sum(mask, axis=1, keepdims=True)
        out_ref[...] = gsum / jnp.maximum(cnt, 1.0) + b2_ref[...]

    return pl.pallas_call(
        body,
        out_shape=jax.ShapeDtypeStruct((GP, 16), jnp.float32),
    )(agg2, zp, deg2, batch2d, b2p)


def kernel(x, edge_index, batch, W1, b1, W2, b2):
    x = x.astype(jnp.float32)
    src = edge_index[0].astype(jnp.int32)
    dst = edge_index[1].astype(jnp.int32)
    pad = EP - E
    srcp3 = jnp.concatenate(
        [src, jnp.zeros((pad,), jnp.int32)]).reshape(NW, CW, CHUNK)
    dstp4 = jnp.concatenate(
        [dst, jnp.full((pad,), DUMMY, jnp.int32)]).reshape(NW, CW, CHUNK)
    xp = jnp.concatenate([x, jnp.zeros((NP - N, D), jnp.float32)])
    batch2d = jnp.concatenate(
        [batch.astype(jnp.int32), jnp.full((NP - N,), G, jnp.int32)]
    ).reshape(1, NP)
    srcp_s = srcp3.reshape(NS, CWS, CHUNK)
    dstp_s = dstp4.reshape(NS, CWS, CHUNK)
    ones_chunk = jnp.ones((CHUNK, 16), jnp.float32)
    zeros16 = jnp.zeros((RPT, 16), jnp.float32)
    zeros64 = jnp.zeros((RPT, HW), jnp.float32)
    W2p = jnp.concatenate(
        [W2.astype(jnp.float32), jnp.zeros((D, 16 - OUT), jnp.float32)], axis=1)
    b1r = b1.astype(jnp.float32).reshape(1, D)
    b2p = jnp.concatenate(
        [b2.astype(jnp.float32), jnp.zeros((16 - OUT,), jnp.float32)]
    ).reshape(1, 16)

    deg2 = _sc_degree(dstp4, ones_chunk, zeros16)
    h1p = _tc_h1(xp, W1.astype(jnp.float32), deg2)
    tabs = h1p.reshape(NP, NC, HW).transpose(1, 0, 2)
    agg = _sc_scatter_feat(tabs, srcp_s, dstp_s, zeros64)
    zp = _tc_z(agg, h1p, deg2, b1r, W2p)
    agg2 = _sc_scatter_rows(zp, srcp3, dstp4, zeros16, 16)
    g = _tc_pool(agg2, zp, deg2, batch2d, b2p)
    g5 = g[:G, :OUT]
    return (g5[:, :2], g5[:, 2:4], g5[:, 4:5])


# speedup vs baseline: 17.3464x; 1.0537x over previous
import jax
import jax.numpy as jnp
from jax import lax
from jax.experimental import pallas as pl
from jax.experimental.pallas import tpu as pltpu
from jax.experimental.pallas import tpu_sc as plsc

N = 10000
E = 320000
D = 128
OUT = 5
G = 20

NC = 2
NS = 16
NW = NC * NS
CHUNK = 128
CW = 80
EP = NW * CW * CHUNK
NP = 10240
DUMMY = N
RPT = NP // NS
GP = 32
BLK = 1024
NB = 4
NBW = 8
HW = 64
CWS = EP // (NS * CHUNK)


def _mesh():
    return plsc.VectorSubcoreMesh(
        core_axis_name="c", subcore_axis_name="s",
        num_cores=NC, num_subcores=NS)


def _sc_degree(dstp4, ones_chunk, zeros_w):
    def body(dst_hbm, ones_hbm, zeros_hbm, out_hbm, didx_v, ones_v, acc_sh):
        c = lax.axis_index("c")
        s = lax.axis_index("s")
        wid = s * NC + c
        t0 = s * RPT
        pltpu.sync_copy(zeros_hbm, acc_sh.at[pl.ds(t0, RPT)])
        pltpu.sync_copy(ones_hbm, ones_v)
        pltpu.sync_copy(dst_hbm.at[wid], didx_v)
        plsc.subcore_barrier()

        def step(k, carry):
            pltpu.sync_copy(ones_v, acc_sh.at[didx_v.at[k]], add=True)
            return carry

        lax.fori_loop(0, CW, step, 0)
        plsc.subcore_barrier()
        pltpu.sync_copy(acc_sh.at[pl.ds(t0, RPT)],
                        out_hbm.at[c, pl.ds(t0, RPT)])

    return pl.kernel(
        body,
        out_type=jax.ShapeDtypeStruct((NC, NP, 16), jnp.float32),
        mesh=_mesh(),
        compiler_params=pltpu.CompilerParams(use_tc_tiling_on_sc=False),
        scratch_types=[
            pltpu.VMEM((CW, CHUNK), jnp.int32),
            pltpu.VMEM((CHUNK, 16), jnp.float32),
            pltpu.VMEM_SHARED((NP, 16), jnp.float32),
        ],
    )(dstp4, ones_chunk, zeros_w)


def _sc_scatter_feat(tabs, srcp_s, dstp_s, zeros_w):
    def body(tab_hbm, src_hbm, dst_hbm, zeros_hbm, out_hbm,
             sidx_v, didx_g, rows_bufs, acc_sh, gsems, ssems):
        c = lax.axis_index("c")
        s = lax.axis_index("s")
        t0 = s * RPT
        pltpu.sync_copy(zeros_hbm, acc_sh.at[pl.ds(t0, RPT)])
        pltpu.sync_copy(src_hbm.at[s], sidx_v)
        plsc.subcore_barrier()

        def group(g, carry):
            base = g * NBW
            gds = []
            for b in range(NBW):
                gds.append(pltpu.async_copy(
                    tab_hbm.at[c].at[sidx_v.at[base + b]],
                    rows_bufs[b], gsems[b]))
            pltpu.sync_copy(dst_hbm.at[s, pl.ds(base, NBW)], didx_g)
            sds = []
            for b in range(NBW):
                gds[b].wait()
                sds.append(pltpu.async_copy(
                    rows_bufs[b], acc_sh.at[didx_g.at[b]], ssems[b],
                    add=True))
            for sd in sds:
                sd.wait()
            return carry

        lax.fori_loop(0, CWS // NBW, group, 0)
        plsc.subcore_barrier()
        pltpu.sync_copy(acc_sh.at[pl.ds(t0, RPT)],
                        out_hbm.at[c, pl.ds(t0, RPT)])

    return pl.kernel(
        body,
        out_type=jax.ShapeDtypeStruct((NC, NP, HW), jnp.float32),
        mesh=_mesh(),
        compiler_params=pltpu.CompilerParams(use_tc_tiling_on_sc=False),
        scratch_types=[
            pltpu.VMEM((CWS, CHUNK), jnp.int32),
            pltpu.VMEM((NBW, CHUNK), jnp.int32),
            [pltpu.VMEM((CHUNK, HW), jnp.float32) for _ in range(NBW)],
            pltpu.VMEM_SHARED((NP, HW), jnp.float32),
            [pltpu.SemaphoreType.DMA for _ in range(NBW)],
            [pltpu.SemaphoreType.DMA for _ in range(NBW)],
        ],
    )(tabs, srcp_s, dstp_s, zeros_w)


def _sc_scatter_rows(table, srcp3, dstp4, zeros_w, width):
    def body(tab_hbm, src_hbm, dst_hbm, zeros_hbm, out_hbm,
             sidx_v, didx_v, rows_bufs, acc_sh, gsems, ssems):
        c = lax.axis_index("c")
        s = lax.axis_index("s")
        wid = s * NC + c
        t0 = s * RPT
        pltpu.sync_copy(zeros_hbm, acc_sh.at[pl.ds(t0, RPT)])
        pltpu.sync_copy(src_hbm.at[wid], sidx_v)
        pltpu.sync_copy(dst_hbm.at[wid], didx_v)
        plsc.subcore_barrier()

        def group(g, carry):
            base = g * NBW
            gds = []
            for b in range(NBW):
                gds.append(pltpu.async_copy(
                    tab_hbm.at[sidx_v.at[base + b]], rows_bufs[b], gsems[b]))
            sds = []
            for b in range(NBW):
                gds[b].wait()
                sds.append(pltpu.async_copy(
                    rows_bufs[b], acc_sh.at[didx_v.at[base + b]], ssems[b],
                    add=True))
            for sd in sds:
                sd.wait()
            return carry

        lax.fori_loop(0, CW // NBW, group, 0)
        plsc.subcore_barrier()
        pltpu.sync_copy(acc_sh.at[pl.ds(t0, RPT)],
                        out_hbm.at[c, pl.ds(t0, RPT)])

    return pl.kernel(
        body,
        out_type=jax.ShapeDtypeStruct((NC, NP, width), jnp.float32),
        mesh=_mesh(),
        compiler_params=pltpu.CompilerParams(
            use_tc_tiling_on_sc=False) if width < 128 else None,
        scratch_types=[
            pltpu.VMEM((CW, CHUNK), jnp.int32),
            pltpu.VMEM((CW, CHUNK), jnp.int32),
            [pltpu.VMEM((CHUNK, width), jnp.float32) for _ in range(NBW)],
            pltpu.VMEM_SHARED((NP, width), jnp.float32),
            [pltpu.SemaphoreType.DMA for _ in range(NBW)],
            [pltpu.SemaphoreType.DMA for _ in range(NBW)],
        ],
    )(table, srcp3, dstp4, zeros_w)


def _dinv_from(deg_blk):
    return lax.rsqrt(deg_blk[0][:, 0:1] + deg_blk[1][:, 0:1] + 1.0)


def _tc_h1(xp, W1, deg2):
    def body(x_ref, w_ref, deg_ref, out_ref):
        dinv = _dinv_from(deg_ref)
        h = jnp.dot(x_ref[...], w_ref[...],
                    preferred_element_type=jnp.float32)
        out_ref[...] = h * dinv

    return pl.pallas_call(
        body,
        grid=(NP // BLK,),
        in_specs=[
            pl.BlockSpec((BLK, D), lambda i: (i, 0)),
            pl.BlockSpec((D, D), lambda i: (0, 0)),
            pl.BlockSpec((NC, BLK, 16), lambda i: (0, i, 0)),
        ],
        out_specs=pl.BlockSpec((BLK, D), lambda i: (i, 0)),
        out_shape=jax.ShapeDtypeStruct((NP, D), jnp.float32),
    )(xp, W1, deg2)


def _tc_z(agg, h1p, deg2, b1r, W2p):
    def body(agg_ref, h1_ref, deg_ref, b1_ref, w2_ref, out_ref):
        dinv = _dinv_from(deg_ref)
        agg_full = jnp.concatenate([agg_ref[0], agg_ref[1]], axis=1)
        pre = dinv * (agg_full + h1_ref[...]) + b1_ref[...]
        h = jnp.where(pre >= 0, pre, 0.01 * pre)
        z = jnp.dot(h, w2_ref[...], preferred_element_type=jnp.float32)
        out_ref[...] = z * dinv

    return pl.pallas_call(
        body,
        grid=(NP // BLK,),
        in_specs=[
            pl.BlockSpec((NC, BLK, HW), lambda i: (0, i, 0)),
            pl.BlockSpec((BLK, D), lambda i: (i, 0)),
            pl.BlockSpec((NC, BLK, 16), lambda i: (0, i, 0)),
            pl.BlockSpec((1, D), lambda i: (0, 0)),
            pl.BlockSpec((D, 16), lambda i: (0, 0)),
        ],
        out_specs=pl.BlockSpec((BLK, 16), lambda i: (i, 0)),
        out_shape=jax.ShapeDtypeStruct((NP, 16), jnp.float32),
    )(agg, h1p, deg2, b1r, W2p)


def _tc_pool(agg2, zp, deg2, batch2d, b2p):
    def body(agg_ref, zp_ref, deg_ref, b_ref, b2_ref, out_ref):
        dinv = _dinv_from(deg_ref)
        a2 = dinv * (agg_ref[0] + agg_ref[1] + zp_ref[...])
        gids = lax.broadcasted_iota(jnp.int32, (GP, NP), 0)
        mask = (b_ref[...] == gids).astype(jnp.float32)
        gsum = jnp.dot(mask, a2, preferred_element_type=jnp.float32)
        cnt = jnp.sum(mask, axis=1, keepdims=True)
        out_ref[...] = gsum / jnp.maximum(cnt, 1.0) + b2_ref[...]

    return pl.pallas_call(
        body,
        out_shape=jax.ShapeDtypeStruct((GP, 16), jnp.float32),
    )(agg2, zp, deg2, batch2d, b2p)


def kernel(x, edge_index, batch, W1, b1, W2, b2):
    x = x.astype(jnp.float32)
    src = edge_index[0].astype(jnp.int32)
    dst = edge_index[1].astype(jnp.int32)
    pad = EP - E
    srcp3 = jnp.concatenate(
        [src, jnp.zeros((pad,), jnp.int32)]).reshape(NW, CW, CHUNK)
    dstp4 = jnp.concatenate(
        [dst, jnp.full((pad,), DUMMY, jnp.int32)]).reshape(NW, CW, CHUNK)
    xp = jnp.concatenate([x, jnp.zeros((NP - N, D), jnp.float32)])
    batch2d = jnp.concatenate(
        [batch.astype(jnp.int32), jnp.full((NP - N,), G, jnp.int32)]
    ).reshape(1, NP)
    srcp_s = srcp3.reshape(NS, CWS, CHUNK)
    dstp_s = dstp4.reshape(NS, CWS, CHUNK)
    ones_chunk = jnp.ones((CHUNK, 16), jnp.float32)
    zeros16 = jnp.zeros((RPT, 16), jnp.float32)
    zeros64 = jnp.zeros((RPT, HW), jnp.float32)
    W2p = jnp.concatenate(
        [W2.astype(jnp.float32), jnp.zeros((D, 16 - OUT), jnp.float32)], axis=1)
    b1r = b1.astype(jnp.float32).reshape(1, D)
    b2p = jnp.concatenate(
        [b2.astype(jnp.float32), jnp.zeros((16 - OUT,), jnp.float32)]
    ).reshape(1, 16)

    deg2 = _sc_degree(dstp4, ones_chunk, zeros16)
    h1p = _tc_h1(xp, W1.astype(jnp.float32), deg2)
    tabs = h1p.reshape(NP, NC, HW).transpose(1, 0, 2)
    agg = _sc_scatter_feat(tabs, srcp_s, dstp_s, zeros64)
    zp = _tc_z(agg, h1p, deg2, b1r, W2p)
    agg2 = _sc_scatter_rows(zp, srcp3, dstp4, zeros16, 16)
    g = _tc_pool(agg2, zp, deg2, batch2d, b2p)
    g5 = g[:G, :OUT]
    return (g5[:, :2], g5[:, 2:4], g5[:, 4:5])
